# Initial kernel scaffold; baseline (speedup 1.0000x reference)
import functools

import jax
import jax.numpy as jnp
from jax import lax
from jax.experimental import pallas as pl
from jax.experimental.pallas import tpu as pltpu
from jax.experimental.pallas import tpu_sc as plsc

VOCAB = 1000000
EMB = 32
BATCH = 4096
HIST = 200
TOTAL = BATCH * HIST

NC = 2
NS = 16
NW = NC * NS
B_PER_W = TOTAL // NW
CHUNK = 1024
NCHUNK = B_PER_W // CHUNK

_mesh = plsc.VectorSubcoreMesh(
    core_axis_name="c", subcore_axis_name="s", num_cores=NC, num_subcores=NS
)


@functools.partial(
    pl.kernel,
    out_type=jax.ShapeDtypeStruct((TOTAL, EMB), jnp.float32),
    mesh=_mesh,
    scratch_types=[
        pltpu.VMEM((CHUNK,), jnp.int32),
        pltpu.VMEM((CHUNK, EMB), jnp.float32),
        pltpu.SemaphoreType.DMA,
    ],
)
def _emb_lookup(idx_hbm, table_hbm, out_hbm, idx_v, rows_v, sem):
    wid = lax.axis_index("s") * NC + lax.axis_index("c")
    base = wid * B_PER_W

    def step(i, carry):
        off = base + i * CHUNK
        pltpu.sync_copy(idx_hbm.at[pl.ds(off, CHUNK)], idx_v)
        pltpu.async_copy(table_hbm.at[idx_v], rows_v, sem).wait()
        pltpu.sync_copy(rows_v, out_hbm.at[pl.ds(off, CHUNK)])
        return carry

    lax.fori_loop(0, NCHUNK, step, 0)


def kernel(indices, table):
    idx_flat = indices.reshape(TOTAL)
    out = _emb_lookup(idx_flat, table)
    return out.reshape(BATCH, HIST, EMB)


# speedup vs baseline: 1.4584x; 1.4584x over previous
import functools

import jax
import jax.numpy as jnp
from jax import lax
from jax.experimental import pallas as pl
from jax.experimental.pallas import tpu as pltpu
from jax.experimental.pallas import tpu_sc as plsc

VOCAB = 1000000
EMB = 32
BATCH = 4096
HIST = 200
TOTAL = BATCH * HIST

NC = 2
NS = 16
NW = NC * NS
B_PER_W = TOTAL // NW
CHUNK = 1024
NCHUNK = B_PER_W // CHUNK

_mesh = plsc.VectorSubcoreMesh(
    core_axis_name="c", subcore_axis_name="s", num_cores=NC, num_subcores=NS
)


@functools.partial(
    pl.kernel,
    out_type=jax.ShapeDtypeStruct((TOTAL, EMB), jnp.float32),
    mesh=_mesh,
    scratch_types=[
        pltpu.VMEM((CHUNK,), jnp.int32),
        pltpu.VMEM((CHUNK, EMB), jnp.float32),
        pltpu.SemaphoreType.DMA,
    ],
    compiler_params=pltpu.CompilerParams(use_tc_tiling_on_sc=False),
)
def _emb_lookup(idx_hbm, table_hbm, out_hbm, idx_v, rows_v, sem):
    wid = lax.axis_index("s") * NC + lax.axis_index("c")
    base = wid * B_PER_W

    def step(i, carry):
        off = base + i * CHUNK
        pltpu.sync_copy(idx_hbm.at[pl.ds(off, CHUNK)], idx_v)
        pltpu.async_copy(table_hbm.at[idx_v], rows_v, sem).wait()
        pltpu.sync_copy(rows_v, out_hbm.at[pl.ds(off, CHUNK)])
        return carry

    lax.fori_loop(0, NCHUNK, step, 0)


def kernel(indices, table):
    idx_flat = indices.reshape(TOTAL)
    out = _emb_lookup(idx_flat, table)
    return out.reshape(BATCH, HIST, EMB)


# speedup vs baseline: 1.4982x; 1.0273x over previous
import functools

import jax
import jax.numpy as jnp
from jax import lax
from jax.experimental import pallas as pl
from jax.experimental.pallas import tpu as pltpu
from jax.experimental.pallas import tpu_sc as plsc

VOCAB = 1000000
EMB = 32
BATCH = 4096
HIST = 200
TOTAL = BATCH * HIST

NC = 2
NS = 16
NW = NC * NS
B_PER_W = TOTAL // NW
NBUF = 4
CHUNK = 800
NCHUNK = B_PER_W // CHUNK
GROUPS = NCHUNK // NBUF

_mesh = plsc.VectorSubcoreMesh(
    core_axis_name="c", subcore_axis_name="s", num_cores=NC, num_subcores=NS
)


@functools.partial(
    pl.kernel,
    out_type=jax.ShapeDtypeStruct((TOTAL, EMB), jnp.float32),
    mesh=_mesh,
    scratch_types=[
        pltpu.VMEM((NBUF, CHUNK), jnp.int32),
        pltpu.VMEM((NBUF, CHUNK, EMB), jnp.float32),
        pltpu.SemaphoreType.DMA((NBUF,)),
        pltpu.SemaphoreType.DMA((NBUF,)),
        pltpu.SemaphoreType.DMA((NBUF,)),
    ],
    compiler_params=pltpu.CompilerParams(use_tc_tiling_on_sc=False),
)
def _emb_lookup(idx_hbm, table_hbm, out_hbm, idx_v, rows_v, sem_idx, sem_gat, sem_out):
    wid = lax.axis_index("s") * NC + lax.axis_index("c")
    base = wid * B_PER_W

    def idx_copy(g, b):
        off = base + (g * NBUF + b) * CHUNK
        return pltpu.make_async_copy(
            idx_hbm.at[pl.ds(off, CHUNK)], idx_v.at[b], sem_idx.at[b]
        )

    def gather_copy(b):
        return pltpu.make_async_copy(
            table_hbm.at[idx_v.at[b]], rows_v.at[b], sem_gat.at[b]
        )

    def out_copy(g, b):
        off = base + (g * NBUF + b) * CHUNK
        return pltpu.make_async_copy(
            rows_v.at[b], out_hbm.at[pl.ds(off, CHUNK)], sem_out.at[b]
        )

    for b in range(NBUF):
        idx_copy(0, b).start()

    def group(g, carry):
        for b in range(NBUF):
            idx_copy(g, b).wait()

            @pl.when(g > 0)
            def _():
                out_copy(g - 1, b).wait()

            gather_copy(b).start()
        for b in range(NBUF):
            gather_copy(b).wait()
            out_copy(g, b).start()

            @pl.when(g + 1 < GROUPS)
            def _():
                idx_copy(g + 1, b).start()
        return carry

    lax.fori_loop(0, GROUPS, group, 0)
    for b in range(NBUF):
        out_copy(GROUPS - 1, b).wait()


def kernel(indices, table):
    idx_flat = indices.reshape(TOTAL)
    out = _emb_lookup(idx_flat, table)
    return out.reshape(BATCH, HIST, EMB)
